# baseline (speedup 1.0000x reference)
import functools

import jax
import jax.numpy as jnp
from jax import lax
from jax.experimental import pallas as pl
from jax.experimental.pallas import tpu as pltpu
from jax.experimental.pallas import tpu_sc as plsc


def _make_sc_pair_gather(t, dd):
    info = plsc.get_sparse_core_info()
    nw = info.num_cores * info.num_subcores
    tpw = t // nw
    chunk = 80
    assert t % nw == 0 and tpw % chunk == 0
    n_chunks = tpw // chunk

    mesh = plsc.VectorSubcoreMesh(core_axis_name="c", subcore_axis_name="s")

    @functools.partial(
        pl.kernel,
        mesh=mesh,
        out_type=jax.ShapeDtypeStruct((t, dd), jnp.float32),
        scratch_types=[
            pltpu.VMEM((tpw,), jnp.int32),
            pltpu.VMEM((chunk, dd), jnp.float32),
            pltpu.VMEM((chunk, dd), jnp.float32),
            pltpu.SemaphoreType.DMA,
            pltpu.SemaphoreType.DMA,
        ],
    )
    def gather(table_hbm, idx_hbm, out_hbm, idx_v, r0, r1, s0, s1):
        wid = lax.axis_index("s") * info.num_cores + lax.axis_index("c")
        base = wid * tpw
        pltpu.sync_copy(idx_hbm.at[pl.ds(base, tpw)], idx_v)
        rows = (r0, r1)
        sems = (s0, s1)

        def fire(j):
            pltpu.async_copy(
                table_hbm.at[idx_v.at[pl.ds(j * chunk, chunk)]],
                rows[j % 2],
                sems[j % 2],
            )

        fire(0)
        for j in range(n_chunks):
            if j + 1 < n_chunks:
                fire(j + 1)
            pltpu.make_async_copy(
                table_hbm.at[idx_v.at[pl.ds(j * chunk, chunk)]],
                rows[j % 2],
                sems[j % 2],
            ).wait()
            pltpu.sync_copy(
                rows[j % 2], out_hbm.at[pl.ds(base + j * chunk, chunk)]
            )

    return gather


def _make_tc_logits(t, v, d, n, blk=1024):
    assert t % blk == 0

    def body(idx_ref, tgt_ref, log_ref, te_ref, ol_ref):
        idx = idx_ref[0]
        oh = (
            lax.broadcasted_iota(jnp.int32, (blk, v), 1) == idx
        ).astype(jnp.bfloat16)
        te = jnp.dot(oh, tgt_ref[...], preferred_element_type=jnp.float32)
        te_ref[...] = te
        ol_ref[...] = jnp.dot(
            te.astype(jnp.bfloat16), log_ref[...],
            preferred_element_type=jnp.float32,
        )

    return pl.pallas_call(
        body,
        grid=(t // blk,),
        in_specs=[
            pl.BlockSpec((1, blk, 1), lambda i: (i, 0, 0)),
            pl.BlockSpec((v, d), lambda i: (0, 0)),
            pl.BlockSpec((d, n), lambda i: (0, 0)),
        ],
        out_specs=[
            pl.BlockSpec((blk, d), lambda i: (i, 0)),
            pl.BlockSpec((blk, n), lambda i: (i, 0)),
        ],
        out_shape=[
            jax.ShapeDtypeStruct((t, d), jnp.float32),
            jax.ShapeDtypeStruct((t, n), jnp.float32),
        ],
    )


def kernel(source_enumerate, target_enumerate, src_embs, tgt_embs, logits):
    b, s = source_enumerate.shape
    t = b * s
    src_v, d = src_embs.shape
    tgt_v = tgt_embs.shape[0]
    n = logits.shape[1]

    src_idx = source_enumerate.reshape(t).astype(jnp.int32)
    table2 = src_embs.reshape(src_v // 2, 2 * d)
    pairs = _make_sc_pair_gather(t, 2 * d)(table2, src_idx >> 1)
    odd = (src_idx & 1)[:, None].astype(jnp.bool_)
    src_emb = jnp.where(odd, pairs[:, d:], pairs[:, :d]).reshape(b, s, d)

    blk = 1024
    idx3 = target_enumerate.astype(jnp.int32).reshape(t // blk, blk, 1)
    tgt_bf = tgt_embs.astype(jnp.bfloat16)
    log_bf = logits.astype(jnp.bfloat16)
    te, ol = _make_tc_logits(t, tgt_v, d, n, blk)(idx3, tgt_bf, log_bf)

    return (src_emb, te.reshape(b, s, d), ol.reshape(b, s, n))
